# Initial kernel scaffold; baseline (speedup 1.0000x reference)
import jax
import jax.numpy as jnp
from jax.experimental import pallas as pl


def kernel(text, offsets, counts, w_proj, W_fc):
    raise NotImplementedError("write your pallas kernel here")


# speedup vs baseline: 1.3343x; 1.3343x over previous
import functools

import jax
import jax.numpy as jnp
from jax import lax
from jax.experimental import pallas as pl
from jax.experimental.pallas import tpu as pltpu
from jax.experimental.pallas import tpu_sc as plsc

NC, NS, L = 2, 16, 16
NW = NC * NS
N_TOK = 32768
B = 16
C = 4
DOF = 64
TPW = N_TOK // NW
VECS = TPW // L
GCH = 128
NCH = TPW // GCH


def _sc_partials(text2d, offsets, counts, w_proj):
    mesh = plsc.VectorSubcoreMesh(core_axis_name="c", subcore_axis_name="s")

    @functools.partial(
        pl.kernel,
        out_type=jax.ShapeDtypeStruct((NW, B, DOF), jnp.float32),
        mesh=mesh,
        scratch_types=[
            pltpu.VMEM((NCH, GCH), jnp.int32),
            pltpu.VMEM((NCH, GCH), jnp.int32),
            pltpu.VMEM((NCH, GCH, 16), jnp.int32),
            pltpu.VMEM((L, B * DOF), jnp.float32),
            pltpu.VMEM((B, DOF), jnp.float32),
            pltpu.VMEM((B + 1, 16), jnp.int32),
            pltpu.VMEM((C, 16), jnp.float32),
            pltpu.SemaphoreType.DMA,
        ],
        compiler_params=pltpu.CompilerParams(
            needs_layout_passes=False, use_tc_tiling_on_sc=False),
    )
    def k(text_h, off_h, counts_h, w_h, out_h, tok_v, tok4_v, rows_v, acc2,
          accr, off_v, w_v, sem):
        wid = lax.axis_index("s") * NC + lax.axis_index("c")
        base = wid * TPW

        pltpu.sync_copy(text_h.at[pl.ds(wid * NCH, NCH)], tok_v)
        pltpu.sync_copy(off_h, off_v)
        pltpu.sync_copy(w_h, w_v)

        for j in range(NCH):
            for i in range(GCH // 16):
                t = tok_v[j, pl.ds(i * 16, 16)]
                tok4_v[j, pl.ds(i * 16, 16)] = lax.shift_right_logical(t, 2)
        for j in range(NCH):
            pltpu.async_copy(counts_h.at[tok4_v.at[j]], rows_v.at[j], sem).wait()

        lanes = lax.iota(jnp.int32, 16)
        zero16 = jnp.zeros((16,), jnp.float32)

        def bf16_round(v):
            r = plsc.bitcast(v, jnp.int32)
            r = r + 0x7FFF + lax.bitwise_and(
                lax.shift_right_logical(r, 16), jnp.full((16,), 1, jnp.int32))
            r = lax.bitwise_and(r, jnp.full((16,), -65536, jnp.int32))
            return plsc.bitcast(r, jnp.float32)

        wv = [bf16_round(w_v[i, pl.ds(0, 16)]) for i in range(C)]
        offv = [off_v[i, pl.ds(0, 16)] for i in range(B + 1)]

        def init_body(i, carry):
            for l in range(L):
                acc2[l, pl.ds(i * 16, 16)] = zero16
            return carry

        lax.fori_loop(0, VECS, init_body, 0)

        def body(i, carry):
            tok = i * 16 + lanes
            g = base + tok
            ch = lax.shift_right_logical(tok, 7)
            within = lax.bitwise_and(tok, jnp.full((16,), GCH - 1, jnp.int32))
            tv = tok_v[lax.shift_right_logical(i, 3),
                       pl.ds(lax.bitwise_and(i, 7) * 16, 16)]
            word0 = lax.bitwise_and(tv, jnp.full((16,), 3, jnp.int32)) * 4
            cs = [
                plsc.load_gather(
                    rows_v, [ch, within, word0 + k_]
                ).astype(jnp.float32)
                for k_ in range(C)
            ]
            s = cs[0] + cs[1] + cs[2] + cs[3]
            denom = jnp.maximum(s - 1.0, 0.0) + 1.0
            xs = [bf16_round(cs[k_] / denom) for k_ in range(C)]
            num = xs[0] * wv[0] + xs[1] * wv[1] + xs[2] * wv[2] + xs[3] * wv[3]
            u = num * float(DOF - 1)
            j0 = jnp.clip(u.astype(jnp.int32), 0, DOF - 1)
            j1 = jnp.minimum(j0 + 1, DOF - 1)
            w0 = jnp.maximum(1.0 - jnp.abs(u - j0.astype(jnp.float32)), 0.0)
            w1 = jnp.maximum(1.0 - jnp.abs(u - j1.astype(jnp.float32)), 0.0)
            w1 = jnp.where(j1 > j0, w1, 0.0)
            bag = jnp.zeros((16,), jnp.int32)
            for k_ in range(1, B):
                bag = bag + jnp.where(g >= offv[k_], 1, 0)
            rowbase = bag * DOF
            plsc.addupdate_scatter(acc2, [lanes, rowbase + j0], w0)
            plsc.addupdate_scatter(acc2, [lanes, rowbase + j1], w1)
            return carry

        lax.fori_loop(0, VECS, body, 0)

        for b in range(B):
            lb = offv[b + 1] - offv[b] - 1
            len_f = (jnp.maximum(lb, jnp.zeros((16,), jnp.int32)) + 1
                     ).astype(jnp.float32)
            for cc in range(DOF // 16):
                col = b * DOF + cc * 16
                tot = acc2[0, pl.ds(col, 16)]
                for l in range(1, L):
                    tot = tot + acc2[l, pl.ds(col, 16)]
                accr[b, pl.ds(cc * 16, 16)] = tot / len_f

        pltpu.sync_copy(accr, out_h.at[wid])

    return k(text2d, offsets, counts, w_proj)


def _tc_finish(partials, W_fc):
    def body(p_ref, w_ref, o_ref):
        ssum = jnp.sum(p_ref[...], axis=0)
        o_ref[...] = lax.dot_general(
            ssum, w_ref[...], (((1,), (1,)), ((), ())),
            preferred_element_type=jnp.float32)

    return pl.pallas_call(
        body,
        out_shape=jax.ShapeDtypeStruct((B, C), jnp.float32),
    )(partials, W_fc)


def kernel(text, offsets, counts, w_proj, W_fc):
    text2d = text.reshape(NW * NCH, GCH)
    counts16 = counts.reshape(counts.shape[0] // 4, 16)
    off_bc = jnp.tile(offsets[:, None], (1, 16))
    w_bc = jnp.tile(w_proj[:, None], (1, 16))
    partials = _sc_partials(text2d, off_bc, counts16, w_bc)
    return _tc_finish(partials, W_fc)


# speedup vs baseline: 1.3917x; 1.0430x over previous
import functools

import jax
import jax.numpy as jnp
from jax import lax
from jax.experimental import pallas as pl
from jax.experimental.pallas import tpu as pltpu
from jax.experimental.pallas import tpu_sc as plsc

NC, NS, L = 2, 16, 16
NW = NC * NS
N_TOK = 32768
B = 16
C = 4
DOF = 64
TPW = N_TOK // NW
VECS = TPW // L
GCH = 128
NCH = TPW // GCH


def _sc_partials(text2d, offsets, counts, w_proj):
    mesh = plsc.VectorSubcoreMesh(core_axis_name="c", subcore_axis_name="s")

    @functools.partial(
        pl.kernel,
        out_type=jax.ShapeDtypeStruct((NW, B, DOF), jnp.float32),
        mesh=mesh,
        scratch_types=[
            pltpu.VMEM((NCH, GCH), jnp.int32),
            pltpu.VMEM((NCH, GCH), jnp.int32),
            pltpu.VMEM((NCH, GCH, 16), jnp.int32),
            pltpu.VMEM((L, B * DOF), jnp.float32),
            pltpu.VMEM((B, DOF), jnp.float32),
            pltpu.VMEM((B + 1, 16), jnp.int32),
            pltpu.VMEM((C, 16), jnp.float32),
            pltpu.SemaphoreType.DMA,
        ],
        compiler_params=pltpu.CompilerParams(
            needs_layout_passes=False, use_tc_tiling_on_sc=False),
    )
    def k(text_h, off_h, counts_h, w_h, out_h, tok_v, tok4_v, rows_v, acc2,
          accr, off_v, w_v, sem):
        wid = lax.axis_index("s") * NC + lax.axis_index("c")
        base = wid * TPW

        pltpu.sync_copy(text_h.at[pl.ds(wid * NCH, NCH)], tok_v)
        pltpu.sync_copy(off_h, off_v)
        pltpu.sync_copy(w_h, w_v)

        for j in range(NCH):
            for i in range(GCH // 16):
                t = tok_v[j, pl.ds(i * 16, 16)]
                tok4_v[j, pl.ds(i * 16, 16)] = lax.shift_right_logical(t, 2)
        copies = [
            pltpu.async_copy(counts_h.at[tok4_v.at[j]], rows_v.at[j], sem)
            for j in range(NCH)
        ]
        for cp_ in copies:
            cp_.wait()

        lanes = lax.iota(jnp.int32, 16)
        zero16 = jnp.zeros((16,), jnp.float32)

        def bf16_round(v):
            r = plsc.bitcast(v, jnp.int32)
            r = r + 0x7FFF + lax.bitwise_and(
                lax.shift_right_logical(r, 16), jnp.full((16,), 1, jnp.int32))
            r = lax.bitwise_and(r, jnp.full((16,), -65536, jnp.int32))
            return plsc.bitcast(r, jnp.float32)

        wv = [bf16_round(w_v[i, pl.ds(0, 16)]) for i in range(C)]
        offv = [off_v[i, pl.ds(0, 16)] for i in range(B + 1)]

        def init_body(i, carry):
            for l in range(L):
                acc2[l, pl.ds(i * 16, 16)] = zero16
            return carry

        lax.fori_loop(0, VECS, init_body, 0)

        def body(i, carry):
            tok = i * 16 + lanes
            g = base + tok
            ch = lax.shift_right_logical(tok, 7)
            within = lax.bitwise_and(tok, jnp.full((16,), GCH - 1, jnp.int32))
            tv = tok_v[lax.shift_right_logical(i, 3),
                       pl.ds(lax.bitwise_and(i, 7) * 16, 16)]
            word0 = lax.bitwise_and(tv, jnp.full((16,), 3, jnp.int32)) * 4
            cs = [
                plsc.load_gather(
                    rows_v, [ch, within, word0 + k_]
                ).astype(jnp.float32)
                for k_ in range(C)
            ]
            s = cs[0] + cs[1] + cs[2] + cs[3]
            denom = jnp.maximum(s - 1.0, 0.0) + 1.0
            xs = [bf16_round(cs[k_] / denom) for k_ in range(C)]
            num = xs[0] * wv[0] + xs[1] * wv[1] + xs[2] * wv[2] + xs[3] * wv[3]
            u = num * float(DOF - 1)
            j0 = jnp.clip(u.astype(jnp.int32), 0, DOF - 1)
            j1 = jnp.minimum(j0 + 1, DOF - 1)
            w0 = jnp.maximum(1.0 - jnp.abs(u - j0.astype(jnp.float32)), 0.0)
            w1 = jnp.maximum(1.0 - jnp.abs(u - j1.astype(jnp.float32)), 0.0)
            w1 = jnp.where(j1 > j0, w1, 0.0)
            bag = jnp.zeros((16,), jnp.int32)
            for k_ in range(1, B):
                bag = bag + jnp.where(g >= offv[k_], 1, 0)
            rowbase = bag * DOF
            plsc.addupdate_scatter(acc2, [lanes, rowbase + j0], w0)
            plsc.addupdate_scatter(acc2, [lanes, rowbase + j1], w1)
            return carry

        lax.fori_loop(0, VECS, body, 0)

        for b in range(B):
            lb = offv[b + 1] - offv[b] - 1
            len_f = (jnp.maximum(lb, jnp.zeros((16,), jnp.int32)) + 1
                     ).astype(jnp.float32)
            for cc in range(DOF // 16):
                col = b * DOF + cc * 16
                tot = acc2[0, pl.ds(col, 16)]
                for l in range(1, L):
                    tot = tot + acc2[l, pl.ds(col, 16)]
                accr[b, pl.ds(cc * 16, 16)] = tot / len_f

        pltpu.sync_copy(accr, out_h.at[wid])

    return k(text2d, offsets, counts, w_proj)


def _tc_finish(partials, W_fc):
    def body(p_ref, w_ref, o_ref):
        ssum = jnp.sum(p_ref[...], axis=0)
        o_ref[...] = lax.dot_general(
            ssum, w_ref[...], (((1,), (1,)), ((), ())),
            preferred_element_type=jnp.float32)

    return pl.pallas_call(
        body,
        out_shape=jax.ShapeDtypeStruct((B, C), jnp.float32),
    )(partials, W_fc)


def kernel(text, offsets, counts, w_proj, W_fc):
    text2d = text.reshape(NW * NCH, GCH)
    counts16 = counts.reshape(counts.shape[0] // 4, 16)
    off_bc = jnp.tile(offsets[:, None], (1, 16))
    w_bc = jnp.tile(w_proj[:, None], (1, 16))
    partials = _sc_partials(text2d, off_bc, counts16, w_bc)
    return _tc_finish(partials, W_fc)
